# initial kernel scaffold (unmeasured)
import jax
import jax.numpy as jnp
from jax import lax
from jax.experimental import pallas as pl
from jax.experimental.pallas import tpu as pltpu

N_DEV = 4


def kernel(O, Wo):
    B, S, H, D = O.shape
    K = H * D
    M = Wo.shape[1]
    S_out = S // N_DEV
    O3 = O.reshape(B, S, K)

    def body(o_hbm, wo_ref, out_hbm, comm, ostage, load_sem, store_sem,
             send_sems, recv_sems, credit_sem):
        my = lax.axis_index("i")
        left = (my + N_DEV - 1) % N_DEV
        right = (my + 1) % N_DEV

        barrier = pltpu.get_barrier_semaphore()
        for nbr in (left, right):
            pl.semaphore_signal(barrier, inc=1, device_id=(nbr,),
                                device_id_type=pl.DeviceIdType.MESH)
        pl.semaphore_wait(barrier, 2)

        def add_partial(slot, c, accumulate):
            cp = pltpu.make_async_copy(
                o_hbm.at[:, pl.ds(c * S_out, S_out), :], ostage, load_sem)
            cp.start()
            cp.wait()
            for b in range(B):
                p = jnp.dot(ostage[b, :, :], wo_ref[:, :],
                            preferred_element_type=jnp.float32)
                if accumulate:
                    comm[slot, b] = comm[slot, b] + p
                else:
                    comm[slot, b] = p

        add_partial(0, (my + N_DEV - 1) % N_DEV, accumulate=False)

        for h in range(N_DEV - 1):
            s = h % 2
            r = (h + 1) % 2
            if h >= 1:
                pl.semaphore_wait(credit_sem, 1)
            rdma = pltpu.make_async_remote_copy(
                src_ref=comm.at[s],
                dst_ref=comm.at[r],
                send_sem=send_sems.at[h],
                recv_sem=recv_sems.at[h],
                device_id=(right,),
                device_id_type=pl.DeviceIdType.MESH,
            )
            rdma.start()
            rdma.wait_send()
            if h < N_DEV - 2:
                pl.semaphore_signal(credit_sem, inc=1, device_id=(left,),
                                    device_id_type=pl.DeviceIdType.MESH)
            rdma.wait_recv()
            add_partial(r, (my + 2 - h) % N_DEV, accumulate=True)

        st = pltpu.make_async_copy(comm.at[(N_DEV - 1) % 2], out_hbm, store_sem)
        st.start()
        st.wait()

    return pl.pallas_call(
        body,
        out_shape=jax.ShapeDtypeStruct((B, S_out, M), jnp.float32),
        in_specs=[
            pl.BlockSpec(memory_space=pltpu.ANY),
            pl.BlockSpec(memory_space=pltpu.VMEM),
        ],
        out_specs=pl.BlockSpec(memory_space=pltpu.ANY),
        scratch_shapes=[
            pltpu.VMEM((2, B, S_out, M), jnp.float32),
            pltpu.VMEM((B, S_out, K), jnp.float32),
            pltpu.SemaphoreType.DMA,
            pltpu.SemaphoreType.DMA,
            pltpu.SemaphoreType.DMA((N_DEV - 1,)),
            pltpu.SemaphoreType.DMA((N_DEV - 1,)),
            pltpu.SemaphoreType.REGULAR,
        ],
        compiler_params=pltpu.CompilerParams(
            collective_id=0,
            vmem_limit_bytes=64 * 1024 * 1024,
        ),
    )(O3, Wo)


# baseline (device time: 636607 ns/iter reference)
import jax
import jax.numpy as jnp
from jax import lax
from jax.experimental import pallas as pl
from jax.experimental.pallas import tpu as pltpu

N_DEV = 4


def kernel(O, Wo):
    B, S, H, D = O.shape
    K = H * D
    M = Wo.shape[1]
    S_out = S // N_DEV
    O3 = O.reshape(B, S, K)

    def body(o_hbm, wo_ref, out_hbm, comm, ostage, load_sem, store_sem,
             send_sems, recv_sems, credit_sem):
        my = lax.axis_index("i")
        left = (my + N_DEV - 1) % N_DEV
        right = (my + 1) % N_DEV

        barrier = pltpu.get_barrier_semaphore()
        for nbr in (left, right):
            pl.semaphore_signal(barrier, inc=1, device_id=(nbr,),
                                device_id_type=pl.DeviceIdType.MESH)
        pl.semaphore_wait(barrier, 2)

        def add_partial(slot, c, accumulate):
            cp = pltpu.make_async_copy(
                o_hbm.at[:, pl.ds(c * S_out, S_out), :], ostage, load_sem)
            cp.start()
            cp.wait()
            for b in range(B):
                p = jnp.dot(ostage[b, :, :], wo_ref[:, :],
                            preferred_element_type=jnp.float32)
                if accumulate:
                    comm[slot, b] = comm[slot, b] + p
                else:
                    comm[slot, b] = p

        add_partial(0, (my + N_DEV - 1) % N_DEV, accumulate=False)

        for h in range(N_DEV - 1):
            s = h % 2
            r = (h + 1) % 2
            if h >= 1:
                pl.semaphore_wait(credit_sem, 1)
            rdma = pltpu.make_async_remote_copy(
                src_ref=comm.at[s],
                dst_ref=comm.at[r],
                send_sem=send_sems.at[h],
                recv_sem=recv_sems.at[h],
                device_id=(right,),
                device_id_type=pl.DeviceIdType.MESH,
            )
            rdma.start()
            rdma.wait_send()
            if h < N_DEV - 2:
                pl.semaphore_signal(credit_sem, inc=1, device_id=(left,),
                                    device_id_type=pl.DeviceIdType.MESH)
            rdma.wait_recv()
            add_partial(r, (my + 2 - h) % N_DEV, accumulate=True)

        st = pltpu.make_async_copy(comm.at[(N_DEV - 1) % 2], out_hbm, store_sem)
        st.start()
        st.wait()

    return pl.pallas_call(
        body,
        out_shape=jax.ShapeDtypeStruct((B, S_out, M), jnp.float32),
        in_specs=[
            pl.BlockSpec(memory_space=pl.ANY),
            pl.BlockSpec(memory_space=pltpu.VMEM),
        ],
        out_specs=pl.BlockSpec(memory_space=pl.ANY),
        scratch_shapes=[
            pltpu.VMEM((2, B, S_out, M), jnp.float32),
            pltpu.VMEM((B, S_out, K), jnp.float32),
            pltpu.SemaphoreType.DMA,
            pltpu.SemaphoreType.DMA,
            pltpu.SemaphoreType.DMA((N_DEV - 1,)),
            pltpu.SemaphoreType.DMA((N_DEV - 1,)),
            pltpu.SemaphoreType.REGULAR,
        ],
        compiler_params=pltpu.CompilerParams(
            collective_id=0,
            vmem_limit_bytes=64 * 1024 * 1024,
        ),
    )(O3, Wo)


# device time: 365136 ns/iter; 1.7435x vs baseline; 1.7435x over previous
import jax
import jax.numpy as jnp
from jax import lax
from jax.experimental import pallas as pl
from jax.experimental.pallas import tpu as pltpu

N_DEV = 4


def kernel(O, Wo):
    B, S, H, D = O.shape
    K = H * D
    M = Wo.shape[1]
    S_out = S // N_DEV
    O3 = O.reshape(B, S, K).astype(jnp.bfloat16)
    Wob = Wo.astype(jnp.bfloat16)

    def body(o_hbm, wo_ref, out_hbm, comm, ostage, out_vmem, load_sem,
             store_sem, send_sems, recv_sems, credit_sem):
        my = lax.axis_index("i")
        left = (my + N_DEV - 1) % N_DEV
        right = (my + 1) % N_DEV

        barrier = pltpu.get_barrier_semaphore()
        for nbr in (left, right):
            pl.semaphore_signal(barrier, inc=1, device_id=(nbr,),
                                device_id_type=pl.DeviceIdType.MESH)
        pl.semaphore_wait(barrier, 2)

        def add_partial(slot, c, mode):
            cp = pltpu.make_async_copy(
                o_hbm.at[:, pl.ds(c * S_out, S_out), :], ostage, load_sem)
            cp.start()
            cp.wait()
            for b in range(B):
                p = jnp.dot(ostage[b, :, :], wo_ref[:, :],
                            preferred_element_type=jnp.float32)
                if mode == "init":
                    comm[slot, b] = p.astype(jnp.bfloat16)
                elif mode == "acc":
                    acc = comm[slot, b].astype(jnp.float32) + p
                    comm[slot, b] = acc.astype(jnp.bfloat16)
                else:
                    out_vmem[b, :, :] = comm[slot, b].astype(jnp.float32) + p

        add_partial(0, (my + N_DEV - 1) % N_DEV, mode="init")

        for h in range(N_DEV - 1):
            s = h % 2
            r = (h + 1) % 2
            if h >= 1:
                pl.semaphore_wait(credit_sem, 1)
            rdma = pltpu.make_async_remote_copy(
                src_ref=comm.at[s],
                dst_ref=comm.at[r],
                send_sem=send_sems.at[h],
                recv_sem=recv_sems.at[h],
                device_id=(right,),
                device_id_type=pl.DeviceIdType.MESH,
            )
            rdma.start()
            rdma.wait_send()
            if h < N_DEV - 2:
                pl.semaphore_signal(credit_sem, inc=1, device_id=(left,),
                                    device_id_type=pl.DeviceIdType.MESH)
            rdma.wait_recv()
            add_partial(r, (my + 2 - h) % N_DEV,
                        mode="final" if h == N_DEV - 2 else "acc")

        st = pltpu.make_async_copy(out_vmem, out_hbm, store_sem)
        st.start()
        st.wait()

    return pl.pallas_call(
        body,
        out_shape=jax.ShapeDtypeStruct((B, S_out, M), jnp.float32),
        in_specs=[
            pl.BlockSpec(memory_space=pl.ANY),
            pl.BlockSpec(memory_space=pltpu.VMEM),
        ],
        out_specs=pl.BlockSpec(memory_space=pl.ANY),
        scratch_shapes=[
            pltpu.VMEM((2, B, S_out, M), jnp.bfloat16),
            pltpu.VMEM((B, S_out, K), jnp.bfloat16),
            pltpu.VMEM((B, S_out, M), jnp.float32),
            pltpu.SemaphoreType.DMA,
            pltpu.SemaphoreType.DMA,
            pltpu.SemaphoreType.DMA((N_DEV - 1,)),
            pltpu.SemaphoreType.DMA((N_DEV - 1,)),
            pltpu.SemaphoreType.REGULAR,
        ],
        compiler_params=pltpu.CompilerParams(
            collective_id=0,
            vmem_limit_bytes=64 * 1024 * 1024,
        ),
    )(O3, Wob)


# device time: 335831 ns/iter; 1.8956x vs baseline; 1.0873x over previous
import jax
import jax.numpy as jnp
from jax import lax
from jax.experimental import pallas as pl
from jax.experimental.pallas import tpu as pltpu

N_DEV = 4


def kernel(O, Wo):
    B, S, H, D = O.shape
    K = H * D
    M = Wo.shape[1]
    S_out = S // N_DEV
    O3 = O.reshape(B, S, K).astype(jnp.bfloat16)
    Wob = Wo.astype(jnp.bfloat16)

    def body(o_hbm, wo_ref, out_hbm, comm, ostage, out_vmem, pbuf, load_sem,
             store_sem, send_sems, recv_sems, credit_sem):
        my = lax.axis_index("i")
        left = (my + N_DEV - 1) % N_DEV
        right = (my + 1) % N_DEV

        barrier = pltpu.get_barrier_semaphore()
        for nbr in (left, right):
            pl.semaphore_signal(barrier, inc=1, device_id=(nbr,),
                                device_id_type=pl.DeviceIdType.MESH)
        pl.semaphore_wait(barrier, 2)

        def compute_partial(dst, c):
            cp = pltpu.make_async_copy(
                o_hbm.at[:, pl.ds(c * S_out, S_out), :], ostage, load_sem)
            cp.start()
            cp.wait()
            for b in range(B):
                dst[b, :, :] = jnp.dot(ostage[b, :, :], wo_ref[:, :],
                                       preferred_element_type=jnp.float32)

        compute_partial(pbuf, (my + N_DEV - 1) % N_DEV)
        for b in range(B):
            comm[0, b] = pbuf[b, :, :].astype(jnp.bfloat16)

        for h in range(N_DEV - 1):
            s = h % 2
            r = (h + 1) % 2
            if h >= 1:
                pl.semaphore_wait(credit_sem, 1)
            rdma = pltpu.make_async_remote_copy(
                src_ref=comm.at[s],
                dst_ref=comm.at[r],
                send_sem=send_sems.at[h],
                recv_sem=recv_sems.at[h],
                device_id=(right,),
                device_id_type=pl.DeviceIdType.MESH,
            )
            rdma.start()
            compute_partial(pbuf, (my + 2 - h) % N_DEV)
            rdma.wait_send()
            if h < N_DEV - 2:
                pl.semaphore_signal(credit_sem, inc=1, device_id=(left,),
                                    device_id_type=pl.DeviceIdType.MESH)
            rdma.wait_recv()
            for b in range(B):
                acc = comm[r, b].astype(jnp.float32) + pbuf[b, :, :]
                if h == N_DEV - 2:
                    out_vmem[b, :, :] = acc
                else:
                    comm[r, b] = acc.astype(jnp.bfloat16)

        st = pltpu.make_async_copy(out_vmem, out_hbm, store_sem)
        st.start()
        st.wait()

    return pl.pallas_call(
        body,
        out_shape=jax.ShapeDtypeStruct((B, S_out, M), jnp.float32),
        in_specs=[
            pl.BlockSpec(memory_space=pl.ANY),
            pl.BlockSpec(memory_space=pltpu.VMEM),
        ],
        out_specs=pl.BlockSpec(memory_space=pl.ANY),
        scratch_shapes=[
            pltpu.VMEM((2, B, S_out, M), jnp.bfloat16),
            pltpu.VMEM((B, S_out, K), jnp.bfloat16),
            pltpu.VMEM((B, S_out, M), jnp.float32),
            pltpu.VMEM((B, S_out, M), jnp.float32),
            pltpu.SemaphoreType.DMA,
            pltpu.SemaphoreType.DMA,
            pltpu.SemaphoreType.DMA((N_DEV - 1,)),
            pltpu.SemaphoreType.DMA((N_DEV - 1,)),
            pltpu.SemaphoreType.REGULAR,
        ],
        compiler_params=pltpu.CompilerParams(
            collective_id=0,
            vmem_limit_bytes=64 * 1024 * 1024,
        ),
    )(O3, Wob)


# device time: 203127 ns/iter; 3.1340x vs baseline; 1.6533x over previous
import jax
import jax.numpy as jnp
from jax import lax
from jax.experimental import pallas as pl
from jax.experimental.pallas import tpu as pltpu

N_DEV = 4


def kernel(O, Wo):
    B, S, H, D = O.shape
    K = H * D
    M = Wo.shape[1]
    Mh = M // 2
    S_out = S // N_DEV
    O3 = O.reshape(B, S, K).astype(jnp.bfloat16)
    Wob = Wo.astype(jnp.bfloat16)

    def body(o_hbm, wo_ref, out_hbm, comm_cw, comm_ccw, ostage, pbuf_cw,
             pbuf_ccw, load_sem, store_sems, send_cw, recv_cw, send_ccw,
             recv_ccw, credit_cw, credit_ccw):
        my = lax.axis_index("i")
        left = (my + N_DEV - 1) % N_DEV
        right = (my + 1) % N_DEV

        barrier = pltpu.get_barrier_semaphore()
        for nbr in (left, right):
            pl.semaphore_signal(barrier, inc=1, device_id=(nbr,),
                                device_id_type=pl.DeviceIdType.MESH)
        pl.semaphore_wait(barrier, 2)

        def compute_partials(c_cw, c_ccw):
            for c, dst, lo in ((c_cw, pbuf_cw, 0), (c_ccw, pbuf_ccw, Mh)):
                cp = pltpu.make_async_copy(
                    o_hbm.at[:, pl.ds(c * S_out, S_out), :], ostage, load_sem)
                cp.start()
                cp.wait()
                for b in range(B):
                    dst[b, :, :] = jnp.dot(
                        ostage[b, :, :], wo_ref[:, lo:lo + Mh],
                        preferred_element_type=jnp.float32)

        compute_partials((my + N_DEV - 1) % N_DEV, (my + 1) % N_DEV)
        for b in range(B):
            comm_cw[0, b] = pbuf_cw[b, :, :].astype(jnp.bfloat16)
            comm_ccw[0, b] = pbuf_ccw[b, :, :].astype(jnp.bfloat16)

        for h in range(N_DEV - 1):
            s = h % 2
            r = (h + 1) % 2
            if h >= 1:
                pl.semaphore_wait(credit_cw, 1)
                pl.semaphore_wait(credit_ccw, 1)
            rdma_cw = pltpu.make_async_remote_copy(
                src_ref=comm_cw.at[s],
                dst_ref=comm_cw.at[r],
                send_sem=send_cw.at[h],
                recv_sem=recv_cw.at[h],
                device_id=(right,),
                device_id_type=pl.DeviceIdType.MESH,
            )
            rdma_ccw = pltpu.make_async_remote_copy(
                src_ref=comm_ccw.at[s],
                dst_ref=comm_ccw.at[r],
                send_sem=send_ccw.at[h],
                recv_sem=recv_ccw.at[h],
                device_id=(left,),
                device_id_type=pl.DeviceIdType.MESH,
            )
            rdma_cw.start()
            rdma_ccw.start()
            compute_partials((my + 2 - h) % N_DEV, (my + 2 + h) % N_DEV)
            rdma_cw.wait_send()
            rdma_ccw.wait_send()
            if h < N_DEV - 2:
                pl.semaphore_signal(credit_cw, inc=1, device_id=(left,),
                                    device_id_type=pl.DeviceIdType.MESH)
                pl.semaphore_signal(credit_ccw, inc=1, device_id=(right,),
                                    device_id_type=pl.DeviceIdType.MESH)
            rdma_cw.wait_recv()
            rdma_ccw.wait_recv()
            for comm, pbuf in ((comm_cw, pbuf_cw), (comm_ccw, pbuf_ccw)):
                for b in range(B):
                    acc = comm[r, b].astype(jnp.float32) + pbuf[b, :, :]
                    if h == N_DEV - 2:
                        pbuf[b, :, :] = acc
                    else:
                        comm[r, b] = acc.astype(jnp.bfloat16)

        st0 = pltpu.make_async_copy(
            pbuf_cw, out_hbm.at[:, :, pl.ds(0, Mh)], store_sems.at[0])
        st1 = pltpu.make_async_copy(
            pbuf_ccw, out_hbm.at[:, :, pl.ds(Mh, Mh)], store_sems.at[1])
        st0.start()
        st1.start()
        st0.wait()
        st1.wait()

    return pl.pallas_call(
        body,
        out_shape=jax.ShapeDtypeStruct((B, S_out, M), jnp.float32),
        in_specs=[
            pl.BlockSpec(memory_space=pl.ANY),
            pl.BlockSpec(memory_space=pltpu.VMEM),
        ],
        out_specs=pl.BlockSpec(memory_space=pl.ANY),
        scratch_shapes=[
            pltpu.VMEM((2, B, S_out, Mh), jnp.bfloat16),
            pltpu.VMEM((2, B, S_out, Mh), jnp.bfloat16),
            pltpu.VMEM((B, S_out, K), jnp.bfloat16),
            pltpu.VMEM((B, S_out, Mh), jnp.float32),
            pltpu.VMEM((B, S_out, Mh), jnp.float32),
            pltpu.SemaphoreType.DMA,
            pltpu.SemaphoreType.DMA((2,)),
            pltpu.SemaphoreType.DMA((N_DEV - 1,)),
            pltpu.SemaphoreType.DMA((N_DEV - 1,)),
            pltpu.SemaphoreType.DMA((N_DEV - 1,)),
            pltpu.SemaphoreType.DMA((N_DEV - 1,)),
            pltpu.SemaphoreType.REGULAR,
            pltpu.SemaphoreType.REGULAR,
        ],
        compiler_params=pltpu.CompilerParams(
            collective_id=0,
            vmem_limit_bytes=64 * 1024 * 1024,
        ),
    )(O3, Wob)
